# Initial kernel scaffold; baseline (speedup 1.0000x reference)
import jax
import jax.numpy as jnp
from jax.experimental import pallas as pl


def kernel(predict_h, predict_t, r, ent_embeddings, rel_embeddings):
    raise NotImplementedError("write your pallas kernel here")


# speedup vs baseline: 1.6145x; 1.6145x over previous
import functools

import jax
import jax.numpy as jnp
from jax import lax
from jax.experimental import pallas as pl
from jax.experimental.pallas import tpu as pltpu
from jax.experimental.pallas import tpu_sc as plsc

B = 16384
D = 128
NC = 2
NS = 16
NW = NC * NS
BPW = B // NW
C = 128
NCH = BPW // C
LJ = D // 16


def _permute(x, idx):
    dnums = lax.GatherDimensionNumbers(
        offset_dims=(), collapsed_slice_dims=(0,), start_index_map=(0,))
    return lax.gather(x, idx[:, None], dnums, slice_sizes=(1,),
                      mode=lax.GatherScatterMode.PROMISE_IN_BOUNDS)


def _sc_body(ent_hbm, idx_h_hbm, idx_t_hbm, rel_hbm, out_hbm,
             idx_h_v, idx_t_v, h_v, t_v, rel_v, out_v, sem_h, sem_t):
    wid = lax.axis_index("c") * NS + lax.axis_index("s")
    base = wid * BPW

    pltpu.sync_copy(idx_h_hbm.at[wid], idx_h_v)
    pltpu.sync_copy(idx_t_hbm.at[wid], idx_t_v)
    pltpu.sync_copy(rel_hbm, rel_v)
    rel_regs = [rel_v[pl.ds(16 * j, 16)] for j in range(LJ)]

    lane = lax.iota(jnp.int32, 16)
    rots = [(lane + sh) & 15 for sh in (8, 4, 2, 1)]

    for chunk in range(NCH):
        cp_h = pltpu.make_async_copy(ent_hbm.at[idx_h_v.at[chunk]], h_v, sem_h)
        cp_t = pltpu.make_async_copy(ent_hbm.at[idx_t_v.at[chunk]], t_v, sem_t)
        cp_h.start()
        cp_t.start()
        cp_h.wait()
        cp_t.wait()

        def group_body(g, _, chunk=chunk):
            row0 = g * 16
            scores = jnp.zeros((16,), jnp.float32)
            for k in range(16):
                acc = jnp.zeros((16,), jnp.float32)
                for j in range(LJ):
                    hv = h_v[row0 + k, pl.ds(16 * j, 16)]
                    tv = t_v[row0 + k, pl.ds(16 * j, 16)]
                    acc = acc + hv * tv * rel_regs[j]
                for rot in rots:
                    acc = acc + _permute(acc, rot)
                scores = jnp.where(lane == k, acc, scores)
            out_v[pl.ds(chunk * C + row0, 16)] = scores
            return 0

        lax.fori_loop(0, C // 16, group_body, 0)

    pltpu.sync_copy(out_v, out_hbm.at[pl.ds(base, BPW)])


@functools.partial(jax.jit, static_argnums=())
def _distmult_sc(ent_embeddings, idx_h, idx_t, rel_row):
    mesh = plsc.VectorSubcoreMesh(core_axis_name="c", subcore_axis_name="s")
    fn = pl.kernel(
        _sc_body,
        out_type=jax.ShapeDtypeStruct((B,), jnp.float32),
        mesh=mesh,
        scratch_types=[
            pltpu.VMEM((NCH, C), jnp.int32),
            pltpu.VMEM((NCH, C), jnp.int32),
            pltpu.VMEM((C, D), jnp.float32),
            pltpu.VMEM((C, D), jnp.float32),
            pltpu.VMEM((D,), jnp.float32),
            pltpu.VMEM((BPW,), jnp.float32),
            pltpu.SemaphoreType.DMA,
            pltpu.SemaphoreType.DMA,
        ],
    )
    return fn(ent_embeddings, idx_h, idx_t, rel_row)


def kernel(predict_h, predict_t, r, ent_embeddings, rel_embeddings):
    rel_row = lax.dynamic_index_in_dim(rel_embeddings, r, axis=0, keepdims=False)
    idx_h = predict_h.reshape(NW, NCH, C)
    idx_t = predict_t.reshape(NW, NCH, C)
    return _distmult_sc(ent_embeddings, idx_h, idx_t, rel_row)


# speedup vs baseline: 2.2621x; 1.4011x over previous
import functools

import jax
import jax.numpy as jnp
from jax import lax
from jax.experimental import pallas as pl
from jax.experimental.pallas import tpu as pltpu
from jax.experimental.pallas import tpu_sc as plsc

B = 16384
D = 128
NC = 2
NS = 16
NW = NC * NS
BPW = B // NW
C = 128
NCH = BPW // C
LJ = D // 16


def _permute(x, idx):
    dnums = lax.GatherDimensionNumbers(
        offset_dims=(), collapsed_slice_dims=(0,), start_index_map=(0,))
    return lax.gather(x, idx[:, None], dnums, slice_sizes=(1,),
                      mode=lax.GatherScatterMode.PROMISE_IN_BOUNDS)


def _sc_body(ent_hbm, idx_h_hbm, idx_t_hbm, rel_hbm, out_hbm,
             idx_h_v, idx_t_v, h0, h1, t0, t1, rel_v, out_v,
             sh0, sh1, st0, st1):
    wid = lax.axis_index("c") * NS + lax.axis_index("s")
    base = wid * BPW

    pltpu.sync_copy(idx_h_hbm.at[wid], idx_h_v)
    pltpu.sync_copy(idx_t_hbm.at[wid], idx_t_v)
    pltpu.sync_copy(rel_hbm, rel_v)
    rel_regs = [rel_v[pl.ds(16 * j, 16)] for j in range(LJ)]

    lane = lax.iota(jnp.int32, 16)
    rots = [(lane + sh) & 15 for sh in (8, 4, 2, 1)]

    hbuf, tbuf = [h0, h1], [t0, t1]
    shs, sts = [sh0, sh1], [st0, st1]
    cps = {}

    def issue(c):
        p = c % 2
        cp_h = pltpu.make_async_copy(ent_hbm.at[idx_h_v.at[c]], hbuf[p], shs[p])
        cp_t = pltpu.make_async_copy(ent_hbm.at[idx_t_v.at[c]], tbuf[p], sts[p])
        cp_h.start()
        cp_t.start()
        cps[c] = (cp_h, cp_t)

    issue(0)
    for chunk in range(NCH):
        if chunk + 1 < NCH:
            issue(chunk + 1)
        cp_h, cp_t = cps.pop(chunk)
        cp_h.wait()
        cp_t.wait()
        p = chunk % 2
        hv_ref, tv_ref = hbuf[p], tbuf[p]

        @plsc.parallel_loop(0, C // 16)
        def group_body(g, hv_ref=hv_ref, tv_ref=tv_ref, chunk=chunk):
            row0 = g * 16
            scores = jnp.zeros((16,), jnp.float32)
            for k in range(16):
                acc = jnp.zeros((16,), jnp.float32)
                for j in range(LJ):
                    hv = hv_ref[row0 + k, pl.ds(16 * j, 16)]
                    tv = tv_ref[row0 + k, pl.ds(16 * j, 16)]
                    acc = acc + hv * tv * rel_regs[j]
                for rot in rots:
                    acc = acc + _permute(acc, rot)
                scores = jnp.where(lane == k, acc, scores)
            out_v[pl.ds(chunk * C + row0, 16)] = scores

    pltpu.sync_copy(out_v, out_hbm.at[pl.ds(base, BPW)])


@jax.jit
def _distmult_sc(ent_embeddings, idx_h, idx_t, rel_row):
    mesh = plsc.VectorSubcoreMesh(core_axis_name="c", subcore_axis_name="s")
    fn = pl.kernel(
        _sc_body,
        out_type=jax.ShapeDtypeStruct((B,), jnp.float32),
        mesh=mesh,
        scratch_types=[
            pltpu.VMEM((NCH, C), jnp.int32),
            pltpu.VMEM((NCH, C), jnp.int32),
            pltpu.VMEM((C, D), jnp.float32),
            pltpu.VMEM((C, D), jnp.float32),
            pltpu.VMEM((C, D), jnp.float32),
            pltpu.VMEM((C, D), jnp.float32),
            pltpu.VMEM((D,), jnp.float32),
            pltpu.VMEM((BPW,), jnp.float32),
            pltpu.SemaphoreType.DMA,
            pltpu.SemaphoreType.DMA,
            pltpu.SemaphoreType.DMA,
            pltpu.SemaphoreType.DMA,
        ],
    )
    return fn(ent_embeddings, idx_h, idx_t, rel_row)


def kernel(predict_h, predict_t, r, ent_embeddings, rel_embeddings):
    rel_row = lax.dynamic_index_in_dim(rel_embeddings, r, axis=0, keepdims=False)
    idx_h = predict_h.reshape(NW, NCH, C)
    idx_t = predict_t.reshape(NW, NCH, C)
    return _distmult_sc(ent_embeddings, idx_h, idx_t, rel_row)
